# baseline (speedup 1.0000x reference)
import functools

import jax
import jax.numpy as jnp
from jax import lax
from jax.experimental import pallas as pl
from jax.experimental.pallas import tpu as pltpu
from jax.experimental.pallas import tpu_sc as plsc

_N = 10000
_E = 320000
_K = 27
_C = 128
_EPS = 1e-5

_NS = 16
_CHUNK = 128
_CHUNKS_PER_TILE = -(-(-(-_E // (_NS * _CHUNK))) // 8) * 8
_EPT = _CHUNKS_PER_TILE * _CHUNK
_E_PAD = _EPT * _NS
_N_PAD = 10240
_RPT = _N_PAD // _NS


def _mm_body(x_ref, w_ref, lo_ref, hi_ref):
    y = jnp.dot(x_ref[...], w_ref[0], preferred_element_type=jnp.float32)
    lo_ref[0] = y[:, :_C]
    hi_ref[0] = y[:, _C:]


def _transform(x, w_cat, interpret=False):
    return pl.pallas_call(
        _mm_body,
        grid=(_K,),
        in_specs=[
            pl.BlockSpec((_N, _C), lambda k: (0, 0)),
            pl.BlockSpec((1, _C, 2 * _C), lambda k: (k, 0, 0)),
        ],
        out_specs=[
            pl.BlockSpec((1, _N, _C), lambda k: (k, 0, 0)),
            pl.BlockSpec((1, _N, _C), lambda k: (k, 0, 0)),
        ],
        out_shape=[jax.ShapeDtypeStruct((_K, _N, _C), jnp.float32)] * 2,
        interpret=interpret,
    )(x, w_cat)


def _gidx_body(s_ref, k_ref, o_ref):
    o_ref[...] = k_ref[...] * _N + s_ref[...]


def _gidx(src2, kid2):
    return pl.pallas_call(
        _gidx_body,
        out_shape=jax.ShapeDtypeStruct(src2.shape, jnp.int32),
    )(src2, kid2)


_SUP = 16
_NSUP = _CHUNKS_PER_TILE // _SUP


def _sc_body(gidx2, dst2, lo_tbl, hi_tbl, out_lo, out_hi,
             ie0, de0, ie1, de1, rows0, rows1,
             acc, esem0, esem1, gsem0, gsem1, ssem0, ssem1):
    cid = lax.axis_index("c")
    sid = lax.axis_index("s")
    base = sid * _CHUNKS_PER_TILE
    sets = [(ie0, de0, esem0), (ie1, de1, esem1)]
    rows = [rows0, rows1]
    gsem = [gsem0, gsem1]
    ssem = [ssem0, ssem1]

    def _zr(r, carry):
        def _zv(i, c):
            rows0[r, pl.ds(i * 16, 16)] = jnp.zeros((16,), jnp.float32)
            return c
        return lax.fori_loop(0, _C // 16, _zv, carry)
    lax.fori_loop(0, _CHUNK, _zr, 0)
    for t in range(_RPT // _CHUNK):
        pltpu.sync_copy(rows0, acc.at[pl.ds(sid * _RPT + t * _CHUNK, _CHUNK)])
    plsc.subcore_barrier()

    def _load_edges(b, st):
        row0 = base + b * _SUP
        pltpu.async_copy(gidx2.at[pl.ds(row0, _SUP)], st[0], st[2])
        pltpu.async_copy(dst2.at[pl.ds(row0, _SUP)], st[1], st[2])

    def _wait_edges(b, st):
        row0 = base + b * _SUP
        pltpu.make_async_copy(gidx2.at[pl.ds(row0, _SUP)], st[0], st[2]).wait()
        pltpu.make_async_copy(dst2.at[pl.ds(row0, _SUP)], st[1], st[2]).wait()

    def _run(tbl):
        def _issue_gather(ie, j, q):
            pltpu.async_copy(tbl.at[ie.at[j]], rows[q], gsem[q])

        def _wait_gather(ie, j, q):
            pltpu.make_async_copy(tbl.at[ie.at[j]], rows[q], gsem[q]).wait()

        def _issue_scatter(de, j, q):
            pltpu.async_copy(rows[q], acc.at[de.at[j]], ssem[q], add=True)

        def _drain_scatter(de, j, q):
            pltpu.make_async_copy(rows[q], acc.at[de.at[j]], ssem[q]).wait()

        _load_edges(0, sets[0])
        _wait_edges(0, sets[0])
        _issue_gather(sets[0][0], 0, 0)

        def _block(b, cur, nxt):
            for j in range(_SUP):
                q = j % 2
                g = b * _SUP + j

                @pl.when(g >= 1)
                def _():
                    _drain_scatter(cur[1], max(j - 1, 0), 1 - q)

                if j < _SUP - 1:
                    _issue_gather(cur[0], j + 1, 1 - q)
                else:
                    @pl.when(b + 1 < _NSUP)
                    def _():
                        _issue_gather(nxt[0], 0, 0)

                _wait_gather(cur[0], j, q)
                _issue_scatter(cur[1], j, q)

                if j == 0:
                    @pl.when(b + 1 < _NSUP)
                    def _():
                        _load_edges(b + 1, nxt)
                if j == 2:
                    @pl.when(b + 1 < _NSUP)
                    def _():
                        _wait_edges(b + 1, nxt)

        def _outer(t, carry):
            _block(2 * t, sets[0], sets[1])
            _block(2 * t + 1, sets[1], sets[0])
            return carry
        lax.fori_loop(0, _NSUP // 2, _outer, 0)
        _drain_scatter(sets[1][1], _SUP - 1, 1)

    @pl.when(cid == 0)
    def _():
        _run(lo_tbl)

    @pl.when(cid == 1)
    def _():
        _run(hi_tbl)

    plsc.subcore_barrier()

    @pl.when(cid == 0)
    def _():
        pltpu.sync_copy(acc.at[pl.ds(sid * _RPT, _RPT)],
                        out_lo.at[pl.ds(sid * _RPT, _RPT)])

    @pl.when(cid == 1)
    def _():
        pltpu.sync_copy(acc.at[pl.ds(sid * _RPT, _RPT)],
                        out_hi.at[pl.ds(sid * _RPT, _RPT)])


def _sc_conv(xk_lo, xk_hi, gidx2, dst2):
    f = pl.kernel(
        _sc_body,
        out_type=[jax.ShapeDtypeStruct((_N_PAD, _C), jnp.float32)] * 2,
        mesh=plsc.VectorSubcoreMesh(core_axis_name="c", subcore_axis_name="s"),
        scratch_types=(
            [pltpu.VMEM((_SUP, _CHUNK), jnp.int32)] * 4
            + [pltpu.VMEM((_CHUNK, _C), jnp.float32)] * 2
            + [pltpu.VMEM_SHARED((_N_PAD, _C), jnp.float32)]
            + [pltpu.SemaphoreType.DMA] * 6
        ),
    )
    return f(gidx2, dst2, xk_lo, xk_hi)


def _fuse_body(lo_ref, hi_ref, glo, blo, ghi, bhi, wlo, whi, gf, bf, out_ref):
    def bn_relu(f, g, b):
        m = jnp.mean(f, axis=0, keepdims=True)
        v = jnp.mean((f - m) ** 2, axis=0, keepdims=True)
        return jnp.maximum((f - m) / jnp.sqrt(v + _EPS) * g + b, 0.0)

    ms_lo = bn_relu(lo_ref[...], glo[...], blo[...])
    ms_hi = bn_relu(hi_ref[...], ghi[...], bhi[...])
    fused = (jnp.dot(ms_lo, wlo[...], preferred_element_type=jnp.float32)
             + jnp.dot(ms_hi, whi[...], preferred_element_type=jnp.float32))
    out_ref[...] = bn_relu(fused, gf[...], bf[...])


def _fuse(conv_lo, conv_hi, g_lo, b_lo, g_hi, b_hi, wf_lo, wf_hi, gf, bf,
          interpret=False):
    return pl.pallas_call(
        _fuse_body,
        out_shape=jax.ShapeDtypeStruct((_N, _C), jnp.float32),
        interpret=interpret,
    )(conv_lo, conv_hi, g_lo.reshape(1, -1), b_lo.reshape(1, -1),
      g_hi.reshape(1, -1), b_hi.reshape(1, -1), wf_lo, wf_hi,
      gf.reshape(1, -1), bf.reshape(1, -1))


def kernel(x, edge_index, kernel_ids, W_med, gamma_med, beta_med,
           W_small, gamma_small, beta_small, W_large, gamma_large,
           beta_large, W_fus, gamma_fus, beta_fus):
    w_cat = jnp.concatenate([W_med, W_small, W_large], axis=2)
    g_hi = jnp.concatenate([gamma_small, gamma_large])
    b_hi = jnp.concatenate([beta_small, beta_large])

    src = edge_index[0]
    dst = edge_index[1]
    pad = _E_PAD - _E
    src2 = jnp.concatenate(
        [src, jnp.zeros((pad,), jnp.int32)]).reshape(-1, _CHUNK)
    dst2 = jnp.concatenate(
        [dst, jnp.full((pad,), _N, jnp.int32)]).reshape(-1, _CHUNK)
    kid2 = jnp.concatenate(
        [kernel_ids, jnp.zeros((pad,), jnp.int32)]).reshape(-1, _CHUNK)

    xk_lo, xk_hi = _transform(x, w_cat)
    gidx2 = _gidx(src2, kid2)
    lo_full, hi_full = _sc_conv(xk_lo.reshape(_K * _N, _C),
                                xk_hi.reshape(_K * _N, _C),
                                gidx2, dst2)
    return _fuse(lo_full[:_N], hi_full[:_N], gamma_med, beta_med, g_hi, b_hi,
                 W_fus[:_C], W_fus[_C:], gamma_fus, beta_fus)


# speedup vs baseline: 1.0013x; 1.0013x over previous
import functools

import jax
import jax.numpy as jnp
from jax import lax
from jax.experimental import pallas as pl
from jax.experimental.pallas import tpu as pltpu
from jax.experimental.pallas import tpu_sc as plsc

_N = 10000
_E = 320000
_K = 27
_C = 128
_EPS = 1e-5

_NS = 16
_CHUNK = 128
_CHUNKS_PER_TILE = -(-(-(-_E // (_NS * _CHUNK))) // 8) * 8
_EPT = _CHUNKS_PER_TILE * _CHUNK
_E_PAD = _EPT * _NS
_N_PAD = 10240
_RPT = _N_PAD // _NS


def _mm_body(x_ref, w_ref, lo_ref, hi_ref):
    y = jnp.dot(x_ref[...], w_ref[0], preferred_element_type=jnp.float32)
    lo_ref[0] = y[:, :_C]
    hi_ref[0] = y[:, _C:]


def _transform(x, w_cat, interpret=False):
    return pl.pallas_call(
        _mm_body,
        grid=(_K,),
        in_specs=[
            pl.BlockSpec((_N, _C), lambda k: (0, 0)),
            pl.BlockSpec((1, _C, 2 * _C), lambda k: (k, 0, 0)),
        ],
        out_specs=[
            pl.BlockSpec((1, _N, _C), lambda k: (k, 0, 0)),
            pl.BlockSpec((1, _N, _C), lambda k: (k, 0, 0)),
        ],
        out_shape=[jax.ShapeDtypeStruct((_K, _N, _C), jnp.float32)] * 2,
        interpret=interpret,
    )(x, w_cat)


def _gidx_body(s_ref, k_ref, o_ref):
    o_ref[...] = k_ref[...] * _N + s_ref[...]


def _gidx(src2, kid2):
    return pl.pallas_call(
        _gidx_body,
        out_shape=jax.ShapeDtypeStruct(src2.shape, jnp.int32),
    )(src2, kid2)


_SUP = 8
_NSUP = _CHUNKS_PER_TILE // _SUP


def _sc_body(gidx2, dst2, lo_tbl, hi_tbl, out_lo, out_hi,
             ie0, de0, ie1, de1, rows0, rows1,
             acc, esem0, esem1, gsem0, gsem1, ssem0, ssem1):
    cid = lax.axis_index("c")
    sid = lax.axis_index("s")
    base = sid * _CHUNKS_PER_TILE
    sets = [(ie0, de0, esem0), (ie1, de1, esem1)]
    rows = [rows0, rows1]
    gsem = [gsem0, gsem1]
    ssem = [ssem0, ssem1]

    def _zr(r, carry):
        def _zv(i, c):
            rows0[r, pl.ds(i * 16, 16)] = jnp.zeros((16,), jnp.float32)
            return c
        return lax.fori_loop(0, _C // 16, _zv, carry)
    lax.fori_loop(0, _CHUNK, _zr, 0)
    for t in range(_RPT // _CHUNK):
        pltpu.sync_copy(rows0, acc.at[pl.ds(sid * _RPT + t * _CHUNK, _CHUNK)])
    plsc.subcore_barrier()

    def _load_edges(b, st):
        row0 = base + b * _SUP
        pltpu.async_copy(gidx2.at[pl.ds(row0, _SUP)], st[0], st[2])
        pltpu.async_copy(dst2.at[pl.ds(row0, _SUP)], st[1], st[2])

    def _wait_edges(b, st):
        row0 = base + b * _SUP
        pltpu.make_async_copy(gidx2.at[pl.ds(row0, _SUP)], st[0], st[2]).wait()
        pltpu.make_async_copy(dst2.at[pl.ds(row0, _SUP)], st[1], st[2]).wait()

    def _run(tbl):
        def _issue_gather(ie, j, q):
            pltpu.async_copy(tbl.at[ie.at[j]], rows[q], gsem[q])

        def _wait_gather(ie, j, q):
            pltpu.make_async_copy(tbl.at[ie.at[j]], rows[q], gsem[q]).wait()

        def _issue_scatter(de, j, q):
            pltpu.async_copy(rows[q], acc.at[de.at[j]], ssem[q], add=True)

        def _drain_scatter(de, j, q):
            pltpu.make_async_copy(rows[q], acc.at[de.at[j]], ssem[q]).wait()

        _load_edges(0, sets[0])
        _wait_edges(0, sets[0])
        _issue_gather(sets[0][0], 0, 0)

        def _block(b, cur, nxt):
            for j in range(_SUP):
                q = j % 2
                g = b * _SUP + j

                @pl.when(g >= 1)
                def _():
                    _drain_scatter(cur[1], max(j - 1, 0), 1 - q)

                if j < _SUP - 1:
                    _issue_gather(cur[0], j + 1, 1 - q)
                else:
                    @pl.when(b + 1 < _NSUP)
                    def _():
                        _issue_gather(nxt[0], 0, 0)

                _wait_gather(cur[0], j, q)
                _issue_scatter(cur[1], j, q)

                if j == 0:
                    @pl.when(b + 1 < _NSUP)
                    def _():
                        _load_edges(b + 1, nxt)
                if j == 2:
                    @pl.when(b + 1 < _NSUP)
                    def _():
                        _wait_edges(b + 1, nxt)

        def _outer(t, carry):
            _block(2 * t, sets[0], sets[1])
            _block(2 * t + 1, sets[1], sets[0])
            return carry
        lax.fori_loop(0, _NSUP // 2, _outer, 0)
        _drain_scatter(sets[1][1], _SUP - 1, 1)

    @pl.when(cid == 0)
    def _():
        _run(lo_tbl)

    @pl.when(cid == 1)
    def _():
        _run(hi_tbl)

    plsc.subcore_barrier()

    @pl.when(cid == 0)
    def _():
        pltpu.sync_copy(acc.at[pl.ds(sid * _RPT, _RPT)],
                        out_lo.at[pl.ds(sid * _RPT, _RPT)])

    @pl.when(cid == 1)
    def _():
        pltpu.sync_copy(acc.at[pl.ds(sid * _RPT, _RPT)],
                        out_hi.at[pl.ds(sid * _RPT, _RPT)])


def _sc_conv(xk_lo, xk_hi, gidx2, dst2):
    f = pl.kernel(
        _sc_body,
        out_type=[jax.ShapeDtypeStruct((_N_PAD, _C), jnp.float32)] * 2,
        mesh=plsc.VectorSubcoreMesh(core_axis_name="c", subcore_axis_name="s"),
        scratch_types=(
            [pltpu.VMEM((_SUP, _CHUNK), jnp.int32)] * 4
            + [pltpu.VMEM((_CHUNK, _C), jnp.float32)] * 2
            + [pltpu.VMEM_SHARED((_N_PAD, _C), jnp.float32)]
            + [pltpu.SemaphoreType.DMA] * 6
        ),
    )
    return f(gidx2, dst2, xk_lo, xk_hi)


def _fuse_body(lo_ref, hi_ref, glo, blo, ghi, bhi, wlo, whi, gf, bf, out_ref):
    def bn_relu(f, g, b):
        m = jnp.mean(f, axis=0, keepdims=True)
        v = jnp.mean((f - m) ** 2, axis=0, keepdims=True)
        return jnp.maximum((f - m) / jnp.sqrt(v + _EPS) * g + b, 0.0)

    ms_lo = bn_relu(lo_ref[...], glo[...], blo[...])
    ms_hi = bn_relu(hi_ref[...], ghi[...], bhi[...])
    fused = (jnp.dot(ms_lo, wlo[...], preferred_element_type=jnp.float32)
             + jnp.dot(ms_hi, whi[...], preferred_element_type=jnp.float32))
    out_ref[...] = bn_relu(fused, gf[...], bf[...])


def _fuse(conv_lo, conv_hi, g_lo, b_lo, g_hi, b_hi, wf_lo, wf_hi, gf, bf,
          interpret=False):
    return pl.pallas_call(
        _fuse_body,
        out_shape=jax.ShapeDtypeStruct((_N, _C), jnp.float32),
        interpret=interpret,
    )(conv_lo, conv_hi, g_lo.reshape(1, -1), b_lo.reshape(1, -1),
      g_hi.reshape(1, -1), b_hi.reshape(1, -1), wf_lo, wf_hi,
      gf.reshape(1, -1), bf.reshape(1, -1))


def kernel(x, edge_index, kernel_ids, W_med, gamma_med, beta_med,
           W_small, gamma_small, beta_small, W_large, gamma_large,
           beta_large, W_fus, gamma_fus, beta_fus):
    w_cat = jnp.concatenate([W_med, W_small, W_large], axis=2)
    g_hi = jnp.concatenate([gamma_small, gamma_large])
    b_hi = jnp.concatenate([beta_small, beta_large])

    src = edge_index[0]
    dst = edge_index[1]
    pad = _E_PAD - _E
    src2 = jnp.concatenate(
        [src, jnp.zeros((pad,), jnp.int32)]).reshape(-1, _CHUNK)
    dst2 = jnp.concatenate(
        [dst, jnp.full((pad,), _N, jnp.int32)]).reshape(-1, _CHUNK)
    kid2 = jnp.concatenate(
        [kernel_ids, jnp.zeros((pad,), jnp.int32)]).reshape(-1, _CHUNK)

    xk_lo, xk_hi = _transform(x, w_cat)
    gidx2 = _gidx(src2, kid2)
    lo_full, hi_full = _sc_conv(xk_lo.reshape(_K * _N, _C),
                                xk_hi.reshape(_K * _N, _C),
                                gidx2, dst2)
    return _fuse(lo_full[:_N], hi_full[:_N], gamma_med, beta_med, g_hi, b_hi,
                 W_fus[:_C], W_fus[_C:], gamma_fus, beta_fus)
